# baseline (speedup 1.0000x reference)
import functools

import jax
import jax.numpy as jnp
from jax import lax
from jax.experimental import pallas as pl
from jax.experimental.pallas import tpu as pltpu
from jax.experimental.pallas import tpu_sc as plsc

B = 16384
D = 32
NC, NS, L = 2, 16, 16
NW = NC * NS
BPW = B // NW
GROUPS = BPW // L

_mesh = plsc.VectorSubcoreMesh(
    core_axis_name="c", subcore_axis_name="s", num_cores=NC, num_subcores=NS
)


@functools.partial(
    pl.kernel,
    out_type=jax.ShapeDtypeStruct((B,), jnp.float32),
    mesh=_mesh,
    scratch_types=[
        pltpu.VMEM((BPW,), jnp.int32),
        pltpu.VMEM((BPW,), jnp.int32),
        pltpu.VMEM((BPW,), jnp.int32),
        pltpu.VMEM((BPW,), jnp.int32),
        pltpu.VMEM((BPW,), jnp.int32),
        pltpu.VMEM((BPW, D), jnp.float32),
        pltpu.VMEM((BPW, D), jnp.float32),
        pltpu.VMEM((BPW, D), jnp.float32),
        pltpu.VMEM((BPW, D), jnp.float32),
        pltpu.VMEM((BPW, D), jnp.float32),
        pltpu.VMEM((BPW, 1), jnp.float32),
        pltpu.VMEM((BPW, 1), jnp.float32),
        pltpu.VMEM((BPW,), jnp.float32),
        pltpu.SemaphoreType.DMA,
    ],
    compiler_params=pltpu.CompilerParams(
        needs_layout_passes=False, use_tc_tiling_on_sc=False
    ),
)
def _sc_score(
    user_hbm, item_hbm, auth_hbm, genr_hbm, subj_hbm,
    ub_hbm, ib_hbm, ue_hbm, ie_hbm, ae_hbm, ge_hbm, se_hbm,
    out_hbm,
    idx_u, idx_i, idx_a, idx_g, idx_s,
    rows_u, rows_i, rows_a, rows_g, rows_s,
    bias_u, bias_i, out_v, sem,
):
    wid = lax.axis_index("s") * NC + lax.axis_index("c")
    base = wid * BPW

    pltpu.sync_copy(user_hbm.at[pl.ds(base, BPW)], idx_u)
    pltpu.sync_copy(item_hbm.at[pl.ds(base, BPW)], idx_i)
    pltpu.sync_copy(auth_hbm.at[pl.ds(base, BPW)], idx_a)
    pltpu.sync_copy(genr_hbm.at[pl.ds(base, BPW)], idx_g)
    pltpu.sync_copy(subj_hbm.at[pl.ds(base, BPW)], idx_s)

    cps = [
        pltpu.async_copy(ue_hbm.at[idx_u], rows_u, sem),
        pltpu.async_copy(ie_hbm.at[idx_i], rows_i, sem),
        pltpu.async_copy(ae_hbm.at[idx_a], rows_a, sem),
        pltpu.async_copy(ge_hbm.at[idx_g], rows_g, sem),
        pltpu.async_copy(se_hbm.at[idx_s], rows_s, sem),
        pltpu.async_copy(ub_hbm.at[idx_u], bias_u, sem),
        pltpu.async_copy(ib_hbm.at[idx_i], bias_i, sem),
    ]
    for cp in cps:
        cp.wait()

    lane = lax.iota(jnp.int32, L)

    def group(g, carry):
        r0 = g * L
        rows_idx = r0 + lane
        zero = jnp.zeros((L,), jnp.int32)
        acc = (plsc.load_gather(bias_u, [rows_idx, zero])
               + plsc.load_gather(bias_i, [rows_idx, zero]))
        for f in range(D):
            fv = jnp.full((L,), f, jnp.int32)
            cu = plsc.load_gather(rows_u, [rows_idx, fv])
            ci = plsc.load_gather(rows_i, [rows_idx, fv])
            ca = plsc.load_gather(rows_a, [rows_idx, fv])
            cg = plsc.load_gather(rows_g, [rows_idx, fv])
            cs = plsc.load_gather(rows_s, [rows_idx, fv])
            acc = acc + cu * (ci + ca + cg + cs)
        out_v[pl.ds(r0, L)] = acc
        return carry

    lax.fori_loop(0, GROUPS, group, 0)
    pltpu.sync_copy(out_v, out_hbm.at[pl.ds(base, BPW)])


def kernel(user, item, item_authors_in, item_authors_off, item_genres_in,
           item_genres_off, item_subjects_in, item_subjects_off,
           u_bias_w, i_bias_w, u_embed_w, i_embed_w, a_embed_w, g_embed_w,
           s_embed_w):
    del item_authors_off, item_genres_off, item_subjects_off
    return _sc_score(
        user.astype(jnp.int32),
        item.astype(jnp.int32),
        item_authors_in.astype(jnp.int32),
        item_genres_in.astype(jnp.int32),
        item_subjects_in.astype(jnp.int32),
        u_bias_w,
        i_bias_w,
        u_embed_w, i_embed_w, a_embed_w, g_embed_w, s_embed_w,
    )


# speedup vs baseline: 2.6743x; 2.6743x over previous
import functools

import jax
import jax.numpy as jnp
from jax import lax
from jax.experimental import pallas as pl
from jax.experimental.pallas import tpu as pltpu
from jax.experimental.pallas import tpu_sc as plsc

B = 16384
D = 32
PK = 128 // D
NC, NS, L = 2, 16, 16
NW = NC * NS
BPW = B // NW
C = 128
NCHUNK = BPW // C
GPC = C // L

_mesh = plsc.VectorSubcoreMesh(
    core_axis_name="c", subcore_axis_name="s", num_cores=NC, num_subcores=NS
)


@functools.partial(
    pl.kernel,
    out_type=jax.ShapeDtypeStruct((B,), jnp.float32),
    mesh=_mesh,
    scratch_types=[
        pltpu.VMEM((BPW,), jnp.int32),
        pltpu.VMEM((BPW,), jnp.int32),
        pltpu.VMEM((BPW,), jnp.int32),
        pltpu.VMEM((BPW,), jnp.int32),
        pltpu.VMEM((BPW,), jnp.int32),
        pltpu.VMEM((BPW,), jnp.int32),
        pltpu.VMEM((BPW,), jnp.int32),
        pltpu.VMEM((BPW,), jnp.int32),
        pltpu.VMEM((BPW,), jnp.int32),
        pltpu.VMEM((BPW,), jnp.int32),
        pltpu.VMEM((C, 128), jnp.float32),
        pltpu.VMEM((C, 128), jnp.float32),
        pltpu.VMEM((C, 128), jnp.float32),
        pltpu.VMEM((C, 128), jnp.float32),
        pltpu.VMEM((C, 128), jnp.float32),
        pltpu.VMEM((BPW,), jnp.float32),
        pltpu.VMEM((BPW,), jnp.float32),
        pltpu.VMEM((BPW,), jnp.float32),
        pltpu.SemaphoreType.DMA,
    ],
    compiler_params=pltpu.CompilerParams(
        needs_layout_passes=False, disable_bounds_checks=True
    ),
)
def _sc_score(
    user_hbm, item_hbm, auth_hbm, genr_hbm, subj_hbm,
    userq_hbm, itemq_hbm, authq_hbm, genrq_hbm, subjq_hbm,
    ub_hbm, ib_hbm, ue_hbm, ie_hbm, ae_hbm, ge_hbm, se_hbm,
    out_hbm,
    idx_u, idx_i, idx_a, idx_g, idx_s,
    idx_uq, idx_iq, idx_aq, idx_gq, idx_sq,
    rows_u, rows_i, rows_a, rows_g, rows_s,
    bias_u, bias_i, out_v, sem,
):
    wid = lax.axis_index("s") * NC + lax.axis_index("c")
    base = wid * BPW

    pltpu.sync_copy(user_hbm.at[pl.ds(base, BPW)], idx_u)
    pltpu.sync_copy(item_hbm.at[pl.ds(base, BPW)], idx_i)
    pltpu.sync_copy(auth_hbm.at[pl.ds(base, BPW)], idx_a)
    pltpu.sync_copy(genr_hbm.at[pl.ds(base, BPW)], idx_g)
    pltpu.sync_copy(subj_hbm.at[pl.ds(base, BPW)], idx_s)
    pltpu.sync_copy(userq_hbm.at[pl.ds(base, BPW)], idx_uq)
    pltpu.sync_copy(itemq_hbm.at[pl.ds(base, BPW)], idx_iq)
    pltpu.sync_copy(authq_hbm.at[pl.ds(base, BPW)], idx_aq)
    pltpu.sync_copy(genrq_hbm.at[pl.ds(base, BPW)], idx_gq)
    pltpu.sync_copy(subjq_hbm.at[pl.ds(base, BPW)], idx_sq)

    bu_cp = pltpu.async_copy(ub_hbm.at[idx_u], bias_u, sem)
    bi_cp = pltpu.async_copy(ib_hbm.at[idx_i], bias_i, sem)
    bu_cp.wait()
    bi_cp.wait()

    lane = lax.iota(jnp.int32, L)

    for k in range(NCHUNK):
        o = k * C
        cps = [
            pltpu.async_copy(ue_hbm.at[idx_uq.at[pl.ds(o, C)]], rows_u, sem),
            pltpu.async_copy(ie_hbm.at[idx_iq.at[pl.ds(o, C)]], rows_i, sem),
            pltpu.async_copy(ae_hbm.at[idx_aq.at[pl.ds(o, C)]], rows_a, sem),
            pltpu.async_copy(ge_hbm.at[idx_gq.at[pl.ds(o, C)]], rows_g, sem),
            pltpu.async_copy(se_hbm.at[idx_sq.at[pl.ds(o, C)]], rows_s, sem),
        ]
        for cp in cps:
            cp.wait()

        def group(g, carry):
            rows_idx = g * L + lane
            sl = pl.ds(o + g * L, L)
            cu_off = (idx_u[sl] & (PK - 1)) * D
            ci_off = (idx_i[sl] & (PK - 1)) * D
            ca_off = (idx_a[sl] & (PK - 1)) * D
            cg_off = (idx_g[sl] & (PK - 1)) * D
            cs_off = (idx_s[sl] & (PK - 1)) * D
            acc = bias_u[sl] + bias_i[sl]
            for f in range(D):
                cu = plsc.load_gather(rows_u, [rows_idx, cu_off + f])
                ci = plsc.load_gather(rows_i, [rows_idx, ci_off + f])
                ca = plsc.load_gather(rows_a, [rows_idx, ca_off + f])
                cg = plsc.load_gather(rows_g, [rows_idx, cg_off + f])
                cs = plsc.load_gather(rows_s, [rows_idx, cs_off + f])
                acc = acc + cu * (ci + ca + cg + cs)
            out_v[sl] = acc
            return carry

        lax.fori_loop(0, GPC, group, 0)

    pltpu.sync_copy(out_v, out_hbm.at[pl.ds(base, BPW)])


def kernel(user, item, item_authors_in, item_authors_off, item_genres_in,
           item_genres_off, item_subjects_in, item_subjects_off,
           u_bias_w, i_bias_w, u_embed_w, i_embed_w, a_embed_w, g_embed_w,
           s_embed_w):
    del item_authors_off, item_genres_off, item_subjects_off
    u32 = user.astype(jnp.int32)
    i32 = item.astype(jnp.int32)
    a32 = item_authors_in.astype(jnp.int32)
    g32 = item_genres_in.astype(jnp.int32)
    s32 = item_subjects_in.astype(jnp.int32)
    return _sc_score(
        u32, i32, a32, g32, s32,
        u32 // PK, i32 // PK, a32 // PK, g32 // PK, s32 // PK,
        u_bias_w.reshape(-1),
        i_bias_w.reshape(-1),
        u_embed_w.reshape(-1, 128), i_embed_w.reshape(-1, 128),
        a_embed_w.reshape(-1, 128), g_embed_w.reshape(-1, 128),
        s_embed_w.reshape(-1, 128),
    )


# speedup vs baseline: 2.9773x; 1.1133x over previous
import functools

import jax
import jax.numpy as jnp
from jax import lax
from jax.experimental import pallas as pl
from jax.experimental.pallas import tpu as pltpu
from jax.experimental.pallas import tpu_sc as plsc

B = 16384
D = 32
TR = 8
NC, NS, L = 2, 16, 16
NW = NC * NS
BPW = B // NW
C = L
NCHUNK = BPW // C

_mesh = plsc.VectorSubcoreMesh(
    core_axis_name="c", subcore_axis_name="s", num_cores=NC, num_subcores=NS
)


@functools.partial(
    pl.kernel,
    out_type=jax.ShapeDtypeStruct((B,), jnp.float32),
    mesh=_mesh,
    scratch_types=[
        pltpu.VMEM((BPW,), jnp.int32),
        pltpu.VMEM((BPW,), jnp.int32),
        pltpu.VMEM((BPW,), jnp.int32),
        pltpu.VMEM((BPW,), jnp.int32),
        pltpu.VMEM((BPW,), jnp.int32),
        pltpu.VMEM((C, TR, D), jnp.float32),
        pltpu.VMEM((C, TR, D), jnp.float32),
        pltpu.VMEM((C, TR, D), jnp.float32),
        pltpu.VMEM((C, TR, D), jnp.float32),
        pltpu.VMEM((C, TR, D), jnp.float32),
        pltpu.VMEM((BPW,), jnp.float32),
        pltpu.VMEM((BPW,), jnp.float32),
        pltpu.VMEM((BPW,), jnp.float32),
        pltpu.SemaphoreType.DMA,
    ],
    compiler_params=pltpu.CompilerParams(
        needs_layout_passes=False, disable_bounds_checks=True
    ),
)
def _sc_score(
    user_hbm, item_hbm, auth_hbm, genr_hbm, subj_hbm,
    ub_hbm, ib_hbm, ue_hbm, ie_hbm, ae_hbm, ge_hbm, se_hbm,
    out_hbm,
    idx_u, idx_i, idx_a, idx_g, idx_s,
    rows_u, rows_i, rows_a, rows_g, rows_s,
    bias_u, bias_i, out_v, sem,
):
    wid = lax.axis_index("s") * NC + lax.axis_index("c")
    base = wid * BPW

    vue = ue_hbm.reshape(ue_hbm.shape[0] // TR, TR, D)
    vie = ie_hbm.reshape(ie_hbm.shape[0] // TR, TR, D)
    vae = ae_hbm.reshape(ae_hbm.shape[0] // TR, TR, D)
    vge = ge_hbm.reshape(ge_hbm.shape[0] // TR, TR, D)
    vse = se_hbm.reshape(se_hbm.shape[0] // TR, TR, D)

    pltpu.sync_copy(user_hbm.at[pl.ds(base, BPW)], idx_u)
    pltpu.sync_copy(item_hbm.at[pl.ds(base, BPW)], idx_i)
    pltpu.sync_copy(auth_hbm.at[pl.ds(base, BPW)], idx_a)
    pltpu.sync_copy(genr_hbm.at[pl.ds(base, BPW)], idx_g)
    pltpu.sync_copy(subj_hbm.at[pl.ds(base, BPW)], idx_s)

    bu_cp = pltpu.async_copy(ub_hbm.at[idx_u], bias_u, sem)
    bi_cp = pltpu.async_copy(ib_hbm.at[idx_i], bias_i, sem)
    bu_cp.wait()
    bi_cp.wait()

    lane = lax.iota(jnp.int32, L)

    def chunk(k, carry):
        sl = pl.ds(k * C, C)
        viu = idx_u[sl]
        vii = idx_i[sl]
        via = idx_a[sl]
        vig = idx_g[sl]
        vis = idx_s[sl]
        vqu = viu >> 3
        vqi = vii >> 3
        vqa = via >> 3
        vqg = vig >> 3
        vqs = vis >> 3
        for j in range(C):
            pltpu.async_copy(vue.at[vqu[j]], rows_u.at[j], sem)
            pltpu.async_copy(vie.at[vqi[j]], rows_i.at[j], sem)
            pltpu.async_copy(vae.at[vqa[j]], rows_a.at[j], sem)
            pltpu.async_copy(vge.at[vqg[j]], rows_g.at[j], sem)
            pltpu.async_copy(vse.at[vqs[j]], rows_s.at[j], sem)
        pltpu.make_async_copy(vue.at[pl.ds(0, C)], rows_u, sem).wait()
        pltpu.make_async_copy(vie.at[pl.ds(0, C)], rows_i, sem).wait()
        pltpu.make_async_copy(vae.at[pl.ds(0, C)], rows_a, sem).wait()
        pltpu.make_async_copy(vge.at[pl.ds(0, C)], rows_g, sem).wait()
        pltpu.make_async_copy(vse.at[pl.ds(0, C)], rows_s, sem).wait()

        ru = viu & (TR - 1)
        ri = vii & (TR - 1)
        ra = via & (TR - 1)
        rg = vig & (TR - 1)
        rs = vis & (TR - 1)
        acc = bias_u[sl] + bias_i[sl]
        for f in range(D):
            fv = jnp.full((L,), f, jnp.int32)
            cu = plsc.load_gather(rows_u, [lane, ru, fv])
            ci = plsc.load_gather(rows_i, [lane, ri, fv])
            ca = plsc.load_gather(rows_a, [lane, ra, fv])
            cg = plsc.load_gather(rows_g, [lane, rg, fv])
            cs = plsc.load_gather(rows_s, [lane, rs, fv])
            acc = acc + cu * (ci + ca + cg + cs)
        out_v[sl] = acc
        return carry

    lax.fori_loop(0, NCHUNK, chunk, 0)

    pltpu.sync_copy(out_v, out_hbm.at[pl.ds(base, BPW)])


def kernel(user, item, item_authors_in, item_authors_off, item_genres_in,
           item_genres_off, item_subjects_in, item_subjects_off,
           u_bias_w, i_bias_w, u_embed_w, i_embed_w, a_embed_w, g_embed_w,
           s_embed_w):
    del item_authors_off, item_genres_off, item_subjects_off
    return _sc_score(
        user.astype(jnp.int32),
        item.astype(jnp.int32),
        item_authors_in.astype(jnp.int32),
        item_genres_in.astype(jnp.int32),
        item_subjects_in.astype(jnp.int32),
        u_bias_w.reshape(-1),
        i_bias_w.reshape(-1),
        u_embed_w, i_embed_w, a_embed_w, g_embed_w, s_embed_w,
    )
